# Initial kernel scaffold; baseline (speedup 1.0000x reference)
import jax
import jax.numpy as jnp
from jax.experimental import pallas as pl


def kernel(x, W1, g1, b1, W2, g2, b2, W3, g3, b3, W4, g4, b4, W5, g5, b5, W6, g6, b6, W7, g7, b7, W8):
    raise NotImplementedError("write your pallas kernel here")


# speedup vs baseline: 1.0001x; 1.0001x over previous
import jax
import jax.numpy as jnp
from jax.experimental import pallas as pl

K = 20


def _knn(x, k):
    inner = -2.0 * jnp.einsum('bcn,bcm->bnm', x, x)
    xx = jnp.sum(x ** 2, axis=1, keepdims=True)
    pd = -xx - inner - jnp.transpose(xx, (0, 2, 1))
    idx = jax.lax.top_k(pd, k)[1]
    return idx


def _get_graph_feature(x, k):
    B, C, N = x.shape
    idx = _knn(x, k)
    xt = jnp.transpose(x, (0, 2, 1))
    feature = xt[jnp.arange(B)[:, None, None], idx]
    xr = xt[:, :, None, :]
    feature = jnp.concatenate([feature - xr, jnp.broadcast_to(xr, feature.shape)], axis=3)
    return jnp.transpose(feature, (0, 3, 1, 2))


def _bn(x, g, b):
    axes = tuple(i for i in range(x.ndim) if i != 1)
    m = jnp.mean(x, axis=axes, keepdims=True)
    v = jnp.var(x, axis=axes, keepdims=True)
    sh = [1] * x.ndim
    sh[1] = -1
    return (x - m) / jnp.sqrt(v + 1e-5) * g.reshape(sh) + b.reshape(sh)


def _pconv(x, W):
    return jnp.einsum('oc,bc...->bo...', W, x)


def _final_matmul_kernel(h_ref, w_ref, o_ref):
    o_ref[...] = jnp.dot(h_ref[...], w_ref[...], preferred_element_type=jnp.float32)


def _final_matmul(h, W8):
    return pl.pallas_call(
        _final_matmul_kernel,
        out_shape=jax.ShapeDtypeStruct((h.shape[0], W8.shape[0]), jnp.float32),
    )(h, W8.T)


def kernel(x, W1, g1, b1, W2, g2, b2, W3, g3, b3, W4, g4, b4, W5, g5, b5, W6, g6, b6, W7, g7, b7, W8):
    f = _get_graph_feature(x, K)
    h = jax.nn.relu(_bn(_pconv(f, W1), g1, b1))
    x1 = jnp.max(h, axis=-1)
    f = _get_graph_feature(x1, K)
    h = jax.nn.relu(_bn(_pconv(f, W2), g2, b2))
    x2 = jnp.max(h, axis=-1)
    f = _get_graph_feature(x2, K)
    h = jax.nn.relu(_bn(_pconv(f, W3), g3, b3))
    x3 = jnp.max(h, axis=-1)
    f = _get_graph_feature(x3, K)
    h = jax.nn.relu(_bn(_pconv(f, W4), g4, b4))
    x4 = jnp.max(h, axis=-1)
    xc = jnp.concatenate([x1, x2, x3, x4], axis=1)
    h = jax.nn.relu(_bn(_pconv(xc, W5), g5, b5))
    p1 = jnp.max(h, axis=-1)
    p2 = jnp.mean(h, axis=-1)
    h = jnp.concatenate([p1, p2], axis=1)
    h = jax.nn.relu(_bn(h @ W6.T, g6, b6))
    h = jax.nn.relu(_bn(h @ W7.T, g7, b7))
    return _final_matmul(h, W8)


# speedup vs baseline: 7.5264x; 7.5258x over previous
import functools

import jax
import jax.numpy as jnp
from jax import lax
from jax.experimental import pallas as pl
from jax.experimental.pallas import tpu as pltpu
from jax.experimental.pallas import tpu_sc as plsc

K = 20
EPS = 1e-5
B = 32
N = 1024
NEG = -3.0e38


def _row_topk(row_load, base):
    iota = lax.iota(jnp.int32, 16)
    neg = jnp.full((16,), NEG, jnp.float32)

    v0 = row_load(0)
    v1 = row_load(1)
    t1 = jnp.maximum(v0, v1)
    t2 = jnp.minimum(v0, v1)
    for j in range(2, 64):
        v = row_load(j)
        lo = jnp.minimum(t1, v)
        t1 = jnp.maximum(t1, v)
        t2 = jnp.maximum(t2, lo)
    t1s, _ = plsc.sort_key_val(t1, iota, descending=True)
    t2s, _ = plsc.sort_key_val(t2, iota, descending=True)
    t2r = lax.rev(t2s, (0,))
    bot = jnp.minimum(t1s, t2r)
    bots, _ = plsc.sort_key_val(bot, iota, descending=True)
    theta = jnp.max(jnp.where(iota == 3, bots, neg))
    return theta, iota


def _pair_max(va, ia, vb, ib):
    take_a = (va > vb) | ((va == vb) & (ia < ib))
    return jnp.where(take_a, va, vb), jnp.where(take_a, ia, ib)


def _pair_min(va, ia, vb, ib):
    take_a = (va > vb) | ((va == vb) & (ia < ib))
    return jnp.where(take_a, vb, va), jnp.where(take_a, ib, ia)


def _merge32(v1, i1, v2, i2):
    v2r = lax.rev(v2, (0,))
    i2r = lax.rev(i2, (0,))
    hv, hi = _pair_max(v1, i1, v2r, i2r)
    lv, li = _pair_min(v1, i1, v2r, i2r)
    hvs, his = plsc.sort_key_val(hv, hi, descending=True)
    lvs, lis = plsc.sort_key_val(lv, li, descending=True)
    return hvs, his, lvs, lis


def _make_sc_knn_gather(cp):
    info = plsc.get_sparse_core_info()
    nc = info.num_cores
    mesh = plsc.VectorSubcoreMesh(core_axis_name="c", subcore_axis_name="s")
    R = 8
    NBLK = N // R

    @functools.partial(
        pl.kernel, mesh=mesh,
        out_type=jax.ShapeDtypeStruct((B, N * K, cp), jnp.float32),
        compiler_params=pltpu.CompilerParams(needs_layout_passes=False),
        scratch_types=[
            pltpu.VMEM((R, N), jnp.float32),
            pltpu.VMEM((1040,), jnp.float32),
            pltpu.VMEM((1040,), jnp.int32),
            pltpu.VMEM((R * 24 + 16,), jnp.int32),
            pltpu.VMEM((R * K, cp), jnp.float32),
            pltpu.SemaphoreType.DMA,
            pltpu.SemaphoreType.DMA,
        ],
    )
    def sck(pd_hbm, x2d_hbm, xg_hbm, pdbuf, valbuf, idxbuf, idxblk, stage,
            sem0, gsem):
        wid = lax.axis_index("s") * nc + lax.axis_index("c")
        base = wid * N
        iota = lax.iota(jnp.int32, 16)
        neg = jnp.full((16,), NEG, jnp.float32)

        def process_block(blk, carry):
            pltpu.async_copy(pd_hbm.at[wid, pl.ds(blk * R, R), :],
                             pdbuf, sem0)
            pltpu.make_async_copy(pd_hbm.at[wid, pl.ds(0, R), :],
                                  pdbuf, sem0).wait()
            def row_body(r, rcarry):
                def row_load(j, _r=r):
                    return pdbuf[_r, pl.ds(j * 16, 16)]

                theta, _ = _row_topk(row_load, base)

                cnt = jnp.int32(0)
                for j in range(64):
                    v = row_load(j)
                    m = v >= theta
                    mi = jnp.where(m, 1, 0).astype(jnp.int32)
                    cs = plsc.cumsum(mi)
                    pos = cs + (cnt - 1)
                    plsc.store_scatter(valbuf, [pos], v, mask=m)
                    plsc.store_scatter(idxbuf, [pos], iota + (j * 16), mask=m)
                    cnt = cnt + jnp.max(cs)
                valbuf[pl.ds(cnt, 16)] = neg

                c0 = valbuf[pl.ds(0, 16)]
                j0 = idxbuf[pl.ds(0, 16)]
                c1 = valbuf[pl.ds(16, 16)]
                j1 = idxbuf[pl.ds(16, 16)]
                c0s, j0s = plsc.sort_key_val(c0, j0, descending=True)
                c1s, j1s = plsc.sort_key_val(c1, j1, descending=True)
                t1v, t1i, t2v, t2i = _merge32(c0s, j0s, c1s, j1s)

                nch = (cnt + 15) // 16

                def mbody(j, st):
                    a1v, a1i, a2v, a2i = st
                    ck = valbuf[pl.ds(j * 16, 16)]
                    ik = idxbuf[pl.ds(j * 16, 16)]
                    cks, iks = plsc.sort_key_val(ck, ik, descending=True)
                    uv, ui = _pair_max(a2v, a2i, lax.rev(cks, (0,)),
                                       lax.rev(iks, (0,)))
                    uvs, uis = plsc.sort_key_val(uv, ui, descending=True)
                    return _merge32(a1v, a1i, uvs, uis)

                t1v, t1i, t2v, t2i = lax.fori_loop(
                    2, nch, mbody, (t1v, t1i, t2v, t2i))

                roff = r * 24
                idxblk[pl.ds(roff, 16)] = t1i + base
                idxblk[pl.ds(roff + 16, 16)] = jnp.where(
                    iota < 4, t2i + base, iota * 0 + base)
                return rcarry

            lax.fori_loop(0, R, row_body, 0)

            for r in range(R):
                pltpu.async_copy(
                    x2d_hbm.at[idxblk.at[pl.ds(r * 24, K)]],
                    stage.at[pl.ds(r * K, K)], gsem)
            for r in range(R):
                pltpu.make_async_copy(
                    x2d_hbm.at[idxblk.at[pl.ds(r * 24, K)]],
                    stage.at[pl.ds(r * K, K)], gsem).wait()
            pltpu.sync_copy(stage,
                            xg_hbm.at[wid, pl.ds(blk * (R * K), R * K), :])
            return carry

        lax.fori_loop(0, NBLK, process_block, 0)

    return sck


def _sc_knn_gather(pd, x2dp, cp):
    return _make_sc_knn_gather(cp)(pd, x2dp)


def _bf(x):
    return x.astype(jnp.bfloat16)


def _tree_sum(z):
    r = z.shape[0]
    while r % 2 == 0 and r > 8:
        half = r // 2
        z = z[:half] + z[half:]
        r = half
    return jnp.sum(z, axis=0)


def _pd_kernel(x_ref, xcm_ref, pd_ref):
    x = x_ref[0]
    xb = _bf(x)
    g = jax.lax.dot_general(xb, xb, (((1,), (1,)), ((), ())),
                            preferred_element_type=jnp.float32)
    inner = -2.0 * g
    xcm = xcm_ref[0]
    xx = jnp.sum(xcm * xcm, axis=0)
    pd_ref[0] = (-xx[:, None] - inner) - xx[None, :]


def _pd(x_nm):
    b, n, c = x_nm.shape
    x_cm = jnp.transpose(x_nm, (0, 2, 1))
    return pl.pallas_call(
        _pd_kernel,
        grid=(b,),
        in_specs=[
            pl.BlockSpec((1, n, c), lambda i: (i, 0, 0)),
            pl.BlockSpec((1, c, n), lambda i: (i, 0, 0)),
        ],
        out_specs=pl.BlockSpec((1, n, n), lambda i: (i, 0, 0)),
        out_shape=jax.ShapeDtypeStruct((b, n, n), jnp.float32),
    )(x_nm, x_cm)


def _edge_kernel(xg_ref, x_ref, wt_ref, m_ref, st_ref, *, tn):
    bi = pl.program_id(0)
    ni = pl.program_id(1)
    xc = x_ref[0]
    c = xc.shape[1]
    xg = xg_ref[0][:, :c]
    xg3 = jnp.reshape(xg, (tn, K, c))
    d3 = xg3 - xc[:, None, :]
    d = jnp.reshape(d3, (tn * K, c))
    xr = jnp.reshape(jnp.broadcast_to(xc[:, None, :], (tn, K, c)), (tn * K, c))
    f = jnp.concatenate([_bf(d), _bf(xr)], axis=1)
    y = jax.lax.dot_general(f, _bf(wt_ref[...]), (((1,), (0,)), ((), ())),
                            preferred_element_type=jnp.float32)
    cout = y.shape[1]
    m_ref[0] = jnp.max(jnp.reshape(y, (tn, K, cout)), axis=1)

    @pl.when(jnp.logical_and(bi == 0, ni == 0))
    def _():
        st_ref[...] = jnp.zeros_like(st_ref)

    for row, part in ((0, _tree_sum(y)), (1, _tree_sum(y * y))):
        s = st_ref[row, :]
        c = st_ref[row + 2, :]
        yv = part - c
        t = s + yv
        st_ref[row + 2, :] = (t - s) - yv
        st_ref[row, :] = t


def _edge(xg_r, x_nm, WT, tn=64):
    b, n, c = x_nm.shape
    cp = xg_r.shape[2]
    cout = WT.shape[1]
    grid = (b, n // tn)
    return pl.pallas_call(
        functools.partial(_edge_kernel, tn=tn),
        grid=grid,
        in_specs=[
            pl.BlockSpec((1, tn * K, cp), lambda i, j: (i, j, 0)),
            pl.BlockSpec((1, tn, c), lambda i, j: (i, j, 0)),
            pl.BlockSpec((2 * c, cout), lambda i, j: (0, 0)),
        ],
        out_specs=[
            pl.BlockSpec((1, tn, cout), lambda i, j: (i, j, 0)),
            pl.BlockSpec((4, cout), lambda i, j: (0, 0)),
        ],
        out_shape=[
            jax.ShapeDtypeStruct((b, n, cout), jnp.float32),
            jax.ShapeDtypeStruct((4, cout), jnp.float32),
        ],
    )(xg_r, x_nm, WT)


def _affine_kernel(m_ref, st_ref, g_ref, b_ref, o_ref, *, cnt):
    s1 = st_ref[0, :] / cnt
    var = st_ref[1, :] / cnt - s1 * s1
    scale = g_ref[...] / jnp.sqrt(var + EPS)
    o_ref[0] = jnp.maximum((m_ref[0] - s1[None, :]) * scale[None, :]
                           + b_ref[...][None, :], 0.0)


def _affine(M, st, g, bb, cnt):
    b, n, cout = M.shape
    return pl.pallas_call(
        functools.partial(_affine_kernel, cnt=cnt),
        grid=(b,),
        in_specs=[
            pl.BlockSpec((1, n, cout), lambda i: (i, 0, 0)),
            pl.BlockSpec((2, cout), lambda i: (0, 0)),
            pl.BlockSpec((cout,), lambda i: (0,)),
            pl.BlockSpec((cout,), lambda i: (0,)),
        ],
        out_specs=pl.BlockSpec((1, n, cout), lambda i: (i, 0, 0)),
        out_shape=jax.ShapeDtypeStruct((b, n, cout), jnp.float32),
    )(M, st, g, bb)


def _mm5_kernel(x_ref, wt_ref, y_ref, st_ref):
    bi = pl.program_id(0)
    y = jax.lax.dot_general(_bf(x_ref[0]), _bf(wt_ref[...]),
                            (((1,), (0,)), ((), ())),
                            preferred_element_type=jnp.float32)
    y_ref[0] = y

    @pl.when(bi == 0)
    def _():
        st_ref[...] = jnp.zeros_like(st_ref)

    for row, part in ((0, _tree_sum(y)), (1, _tree_sum(y * y))):
        s = st_ref[row, :]
        c = st_ref[row + 2, :]
        yv = part - c
        t = s + yv
        st_ref[row + 2, :] = (t - s) - yv
        st_ref[row, :] = t


def _mm5(xc, W5T):
    b, n, c = xc.shape
    cout = W5T.shape[1]
    return pl.pallas_call(
        _mm5_kernel,
        grid=(b,),
        in_specs=[
            pl.BlockSpec((1, n, c), lambda i: (i, 0, 0)),
            pl.BlockSpec((c, cout), lambda i: (0, 0)),
        ],
        out_specs=[
            pl.BlockSpec((1, n, cout), lambda i: (i, 0, 0)),
            pl.BlockSpec((4, cout), lambda i: (0, 0)),
        ],
        out_shape=[
            jax.ShapeDtypeStruct((b, n, cout), jnp.float32),
            jax.ShapeDtypeStruct((4, cout), jnp.float32),
        ],
    )(xc, W5T)


def _pool_kernel(y_ref, st_ref, g_ref, b_ref, p1_ref, p2_ref, *, cnt, n):
    s1 = st_ref[0, :] / cnt
    var = st_ref[1, :] / cnt - s1 * s1
    scale = g_ref[...] / jnp.sqrt(var + EPS)
    h = jnp.maximum((y_ref[0] - s1[None, :]) * scale[None, :]
                    + b_ref[...][None, :], 0.0)
    p1_ref[0, 0] = jnp.max(h, axis=0)
    p2_ref[0, 0] = jnp.sum(h, axis=0) / n


def _pool(y, st, g, bb, cnt):
    b, n, cout = y.shape
    return pl.pallas_call(
        functools.partial(_pool_kernel, cnt=cnt, n=float(n)),
        grid=(b,),
        in_specs=[
            pl.BlockSpec((1, n, cout), lambda i: (i, 0, 0)),
            pl.BlockSpec((2, cout), lambda i: (0, 0)),
            pl.BlockSpec((cout,), lambda i: (0,)),
            pl.BlockSpec((cout,), lambda i: (0,)),
        ],
        out_specs=[
            pl.BlockSpec((1, 1, cout), lambda i: (i, 0, 0)),
            pl.BlockSpec((1, 1, cout), lambda i: (i, 0, 0)),
        ],
        out_shape=[
            jax.ShapeDtypeStruct((b, 1, cout), jnp.float32),
            jax.ShapeDtypeStruct((b, 1, cout), jnp.float32),
        ],
    )(y, st, g, bb)


def _head_kernel(h_ref, w6_ref, g6_ref, b6_ref, w7_ref, g7_ref, b7_ref,
                 w8_ref, o_ref):
    def bn_relu(y, g, bb):
        m = jnp.mean(y, axis=0)
        v = jnp.mean((y - m[None, :]) ** 2, axis=0)
        return jnp.maximum((y - m[None, :]) / jnp.sqrt(v + EPS)
                           * g[None, :] + bb[None, :], 0.0)

    h = h_ref[...]
    y6 = jax.lax.dot_general(_bf(h), _bf(w6_ref[...]), (((1,), (0,)), ((), ())),
                             preferred_element_type=jnp.float32)
    h6 = bn_relu(y6, g6_ref[...], b6_ref[...])
    y7 = jax.lax.dot_general(_bf(h6), _bf(w7_ref[...]), (((1,), (0,)), ((), ())),
                             preferred_element_type=jnp.float32)
    h7 = bn_relu(y7, g7_ref[...], b7_ref[...])
    o_ref[...] = jax.lax.dot_general(_bf(h7), _bf(w8_ref[...]),
                                     (((1,), (0,)), ((), ())),
                                     preferred_element_type=jnp.float32)


def _head(h, W6T, g6, b6, W7T, g7, b7, W8T):
    return pl.pallas_call(
        _head_kernel,
        out_shape=jax.ShapeDtypeStruct((h.shape[0], W8T.shape[1]), jnp.float32),
    )(h, W6T, g6, b6, W7T, g7, b7, W8T)


def _edge_layer(x_nm, W, g, bb):
    b, n, c = x_nm.shape
    pd = _pd(x_nm)
    cp = 128
    x2d = jnp.reshape(x_nm, (b * n, c))
    if cp != c:
        x2d = jnp.pad(x2d, ((0, 0), (0, cp - c)))
    xg_r = _sc_knn_gather(pd, x2d, cp)
    M, st = _edge(xg_r, x_nm, W.T)
    return _affine(M, st[:2], g, bb, float(b * n * K))


def kernel(x, W1, g1, b1, W2, g2, b2, W3, g3, b3, W4, g4, b4, W5, g5, b5,
           W6, g6, b6, W7, g7, b7, W8):
    x_nm = jnp.transpose(x, (0, 2, 1))
    x1 = _edge_layer(x_nm, W1, g1, b1)
    x2 = _edge_layer(x1, W2, g2, b2)
    x3 = _edge_layer(x2, W3, g3, b3)
    x4 = _edge_layer(x3, W4, g4, b4)
    xc = jnp.concatenate([x1, x2, x3, x4], axis=2)
    y5, st5 = _mm5(xc, W5.T)
    p1, p2 = _pool(y5, st5[:2], g5, b5, float(B * N))
    h = jnp.concatenate([p1[:, 0, :], p2[:, 0, :]], axis=1)
    return _head(h, W6.T, g6, b6, W7.T, g7, b7, W8.T)
